# Initial kernel scaffold; baseline (speedup 1.0000x reference)
import jax
import jax.numpy as jnp
from jax.experimental import pallas as pl


def kernel(x, edge_index, W1, b1, W2, b2):
    raise NotImplementedError("write your pallas kernel here")


# speedup vs baseline: 5.4023x; 5.4023x over previous
import functools

import jax
import jax.numpy as jnp
from jax import lax
from jax.experimental import pallas as pl
from jax.experimental.pallas import tpu as pltpu
from jax.experimental.pallas import tpu_sc as plsc

N_NODES = 10000
N_EDGES = 320000
D = 128

NC = 2
NS = 16
NW = NC * NS

EDGES_PER_TILE = N_EDGES // NW
EDGE_BATCH = 80
N_BATCHES = EDGES_PER_TILE // EDGE_BATCH
N_PAD = 10240
ROWS_PER_TILE = N_PAD // NS


def _sc_aggregate(x, row, col, zeros_blk):
    mesh = plsc.VectorSubcoreMesh(core_axis_name="c", subcore_axis_name="s")

    @functools.partial(
        pl.kernel,
        mesh=mesh,
        out_type=jax.ShapeDtypeStruct((NC, N_PAD, D), jnp.float32),
        scratch_types=[
            pltpu.VMEM((EDGE_BATCH,), jnp.int32),
            pltpu.VMEM((EDGE_BATCH,), jnp.int32),
            pltpu.VMEM((EDGE_BATCH, D), jnp.float32),
            pltpu.VMEM_SHARED((N_PAD, D), jnp.float32),
            pltpu.SemaphoreType.DMA,
        ],
    )
    def k(x_hbm, row_hbm, col_hbm, zeros_hbm, out_hbm, idx_r, idx_c, rows_v, agg, sem):
        c = lax.axis_index("c")
        s = lax.axis_index("s")
        wid = s * NC + c

        pltpu.sync_copy(zeros_hbm, agg.at[pl.ds(s * ROWS_PER_TILE, ROWS_PER_TILE)])
        plsc.subcore_barrier()

        base = pl.multiple_of(wid * EDGES_PER_TILE, 8)

        def body(j, carry):
            off = pl.multiple_of(base + j * EDGE_BATCH, 8)
            pltpu.sync_copy(row_hbm.at[pl.ds(off, EDGE_BATCH)], idx_r)
            pltpu.sync_copy(col_hbm.at[pl.ds(off, EDGE_BATCH)], idx_c)
            pltpu.async_copy(x_hbm.at[idx_r], rows_v, sem).wait()
            pltpu.sync_copy(rows_v, agg.at[idx_c], add=True)
            return carry

        lax.fori_loop(0, N_BATCHES, body, 0)
        plsc.subcore_barrier()

        r0 = s * ROWS_PER_TILE
        pltpu.sync_copy(agg.at[pl.ds(r0, ROWS_PER_TILE)],
                        out_hbm.at[c, pl.ds(r0, ROWS_PER_TILE)])

    return k(x, row, col, zeros_blk)


def _mlp_body(x_ref, p0_ref, p1_ref, w1_ref, b1_ref, w2_ref, b2_ref, o_ref):
    h = x_ref[...] + p0_ref[...] + p1_ref[...]
    h1 = jnp.dot(h, w1_ref[...], preferred_element_type=jnp.float32) + b1_ref[...]
    h1 = jnp.maximum(h1, 0.0)
    o_ref[...] = jnp.dot(h1, w2_ref[...], preferred_element_type=jnp.float32) + b2_ref[...]


def _tc_mlp(x, p0, p1, w1t, b1, w2t, b2):
    block = 2000
    grid = (N_NODES // block,)
    row_spec = pl.BlockSpec((block, D), lambda i: (i, 0))
    full_spec = pl.BlockSpec((D, D), lambda i: (0, 0))
    bias_spec = pl.BlockSpec((1, D), lambda i: (0, 0))
    return pl.pallas_call(
        _mlp_body,
        grid=grid,
        in_specs=[row_spec, row_spec, row_spec, full_spec, bias_spec, full_spec, bias_spec],
        out_specs=row_spec,
        out_shape=jax.ShapeDtypeStruct((N_NODES, D), jnp.float32),
    )(x, p0, p1, w1t, b1, w2t, b2)


@jax.jit
def kernel(x, edge_index, W1, b1, W2, b2):
    row = edge_index[0].astype(jnp.int32)
    col = edge_index[1].astype(jnp.int32)
    zeros_blk = jnp.zeros((ROWS_PER_TILE, D), jnp.float32)
    partials = _sc_aggregate(x, row, col, zeros_blk)
    return _tc_mlp(x, partials[0, :N_NODES], partials[1, :N_NODES],
                   W1.T, b1.reshape(1, D), W2.T, b2.reshape(1, D))
